# baseline (speedup 1.0000x reference)
import functools

import jax
import jax.numpy as jnp
from jax import lax
from jax.experimental import pallas as pl
from jax.experimental.pallas import tpu as pltpu
from jax.experimental.pallas import tpu_sc as plsc

N_NODES = 10000
N_EDGES = 320000
D_IN = 128
D_HID = 16

NC = 2
NS = 16
NW = NC * NS
E_PER_W = N_EDGES // NW
CH = 100
NCH = E_PER_W // CH
NRING = 10
LOOK = 5
NOUT = NCH // NRING
N_PAD = 10240
ZR = N_PAD // NS


def _project_kernel(x_ref, w_ref, o_ref):
    o_ref[...] = jnp.dot(x_ref[...], w_ref[...],
                         preferred_element_type=jnp.float32)


def _mlp_kernel(y_ref, p0_ref, p1_ref, w2_ref, b1_ref, b2_ref, s_ref, o_ref):
    s = s_ref[0, 0]
    h = s * y_ref[...] + (p0_ref[...] + p1_ref[...]) + b1_ref[...]
    h = jnp.maximum(h, 0.0)
    h = jnp.dot(h, w2_ref[...], preferred_element_type=jnp.float32) + b2_ref[...]
    o_ref[...] = jnp.maximum(h, 0.0)


@functools.partial(
    pl.kernel,
    out_type=jax.ShapeDtypeStruct((NC, N_PAD, D_HID), jnp.float32),
    mesh=plsc.VectorSubcoreMesh(core_axis_name="c", subcore_axis_name="s"),
    scratch_types=[
        pltpu.VMEM((NCH, CH), jnp.int32),
        pltpu.VMEM((NCH, CH), jnp.int32),
        pltpu.VMEM((NRING, CH, D_HID), jnp.float32),
        pltpu.VMEM((ZR, D_HID), jnp.float32),
        pltpu.VMEM_SHARED((N_PAD, D_HID), jnp.float32),
        pltpu.SemaphoreType.DMA((NRING,)),
        pltpu.SemaphoreType.DMA((NRING,)),
    ],
    compiler_params=pltpu.CompilerParams(use_tc_tiling_on_sc=False),
)
def _sc_aggregate(y_hbm, src_hbm, dst_hbm, parts_hbm,
                  src_v, dst_v, rows_v, stage_v, acc, gsems, ssems):
    cid = lax.axis_index("c")
    sid = lax.axis_index("s")
    wid = cid * NS + sid

    zrow = jnp.zeros((D_HID,), jnp.float32)

    def zbody(i, carry):
        stage_v[i, :] = zrow
        return carry

    lax.fori_loop(0, ZR, zbody, 0)
    pltpu.sync_copy(stage_v, acc.at[pl.ds(sid * ZR, ZR)])

    pltpu.sync_copy(src_hbm.at[wid], src_v)
    pltpu.sync_copy(dst_hbm.at[wid], dst_v)

    for b in range(LOOK):
        pltpu.async_copy(y_hbm.at[src_v.at[b]], rows_v.at[b], gsems.at[b])
    plsc.subcore_barrier()

    def body(g, carry):
        for b in range(NRING):
            c = g * NRING + b
            b2 = (b + LOOK) % NRING
            fut = c + LOOK
            pltpu.make_async_copy(
                y_hbm.at[src_v.at[c]], rows_v.at[b], gsems.at[b]).wait()
            pltpu.async_copy(rows_v.at[b], acc.at[dst_v.at[c]],
                             ssems.at[b], add=True)

            @pl.when(jnp.logical_and(c >= LOOK, fut < NCH))
            def _():
                pltpu.make_async_copy(
                    rows_v.at[b2], acc.at[dst_v.at[0]], ssems.at[b2]).wait()

            @pl.when(fut < NCH)
            def _():
                pltpu.async_copy(
                    y_hbm.at[src_v.at[fut]], rows_v.at[b2], gsems.at[b2])

        return carry

    lax.fori_loop(0, NOUT, body, 0)

    for b in range(NRING):
        pltpu.make_async_copy(
            rows_v.at[b], acc.at[dst_v.at[0]], ssems.at[b]).wait()

    plsc.subcore_barrier()
    pltpu.sync_copy(acc.at[pl.ds(sid * ZR, ZR)], stage_v)
    pltpu.sync_copy(stage_v, parts_hbm.at[cid, pl.ds(sid * ZR, ZR)])


def kernel(x, edge_index, W1, b1, W2, b2, eps):
    y = pl.pallas_call(
        _project_kernel,
        out_shape=jax.ShapeDtypeStruct((N_NODES, D_HID), jnp.float32),
    )(x, W1)

    src3 = edge_index[0].reshape(NW, NCH, CH)
    dst3 = edge_index[1].reshape(NW, NCH, CH)
    parts = _sc_aggregate(y, src3, dst3)[:, :N_NODES, :]

    scale = (1.0 + eps).reshape(1, 1)
    out = pl.pallas_call(
        _mlp_kernel,
        out_shape=jax.ShapeDtypeStruct((N_NODES, D_HID), jnp.float32),
    )(y, parts[0], parts[1], W2,
      b1.reshape(1, D_HID), b2.reshape(1, D_HID), scale)
    return out


# speedup vs baseline: 1.1192x; 1.1192x over previous
import functools

import jax
import jax.numpy as jnp
from jax import lax
from jax.experimental import pallas as pl
from jax.experimental.pallas import tpu as pltpu
from jax.experimental.pallas import tpu_sc as plsc

N_NODES = 10000
N_EDGES = 320000
D_IN = 128
D_HID = 16

NC = 2
NS = 16
NW = NC * NS
E_PER_W = N_EDGES // NW
CH = 100
NCH = E_PER_W // CH
NBUF = 5
NOUT = NCH // NBUF
N_PAD = 10240
ZR = N_PAD // NS


def _project_kernel(x_ref, w_ref, o_ref):
    o_ref[pl.ds(0, N_NODES), :] = jnp.dot(
        x_ref[...], w_ref[...], preferred_element_type=jnp.float32)
    o_ref[pl.ds(N_NODES, N_PAD - N_NODES), :] = jnp.zeros(
        (N_PAD - N_NODES, D_HID), jnp.float32)


def _mlp_kernel(y_ref, p0_ref, p1_ref, w2_ref, b1_ref, b2_ref, s_ref, o_ref):
    s = s_ref[0, 0]
    h = s * y_ref[...] + (p0_ref[...] + p1_ref[...]) + b1_ref[...]
    h = jnp.maximum(h, 0.0)
    h = jnp.dot(h, w2_ref[...], preferred_element_type=jnp.float32) + b2_ref[...]
    o_ref[...] = jnp.maximum(h, 0.0)


@functools.partial(
    pl.kernel,
    out_type=(jax.ShapeDtypeStruct((N_PAD, D_HID), jnp.float32),
              jax.ShapeDtypeStruct((N_PAD, D_HID), jnp.float32)),
    mesh=plsc.VectorSubcoreMesh(core_axis_name="c", subcore_axis_name="s"),
    scratch_types=[
        pltpu.VMEM((NCH, CH), jnp.int32),
        pltpu.VMEM((NCH, CH), jnp.int32),
        pltpu.VMEM((NBUF, CH, D_HID), jnp.float32),
        pltpu.VMEM((ZR, D_HID), jnp.float32),
        pltpu.VMEM_SHARED((N_PAD, D_HID), jnp.float32),
        pltpu.SemaphoreType.DMA((NBUF,)),
    ],
    compiler_params=pltpu.CompilerParams(use_tc_tiling_on_sc=False),
)
def _sc_aggregate(y_hbm, edges_hbm, p0_hbm, p1_hbm,
                  src_v, dst_v, rows_v, stage_v, acc, sems):
    cid = lax.axis_index("c")
    sid = lax.axis_index("s")
    wid = cid * NS + sid

    zrow = jnp.zeros((D_HID,), jnp.float32)

    def zbody(i, carry):
        stage_v[i, :] = zrow
        return carry

    lax.fori_loop(0, ZR, zbody, 0)
    pltpu.sync_copy(stage_v, acc.at[pl.ds(sid * ZR, ZR)])

    pltpu.sync_copy(edges_hbm.at[0, wid], src_v)
    pltpu.sync_copy(edges_hbm.at[1, wid], dst_v)

    for b in range(NBUF):
        pltpu.async_copy(y_hbm.at[src_v.at[b]], rows_v.at[b], sems.at[b])
    plsc.subcore_barrier()

    def body(g, carry):
        for b in range(NBUF):
            c = g * NBUF + b
            pltpu.make_async_copy(
                y_hbm.at[src_v.at[c]], rows_v.at[b], sems.at[b]).wait()
            pltpu.sync_copy(rows_v.at[b], acc.at[dst_v.at[c]], add=True)

            @pl.when(g < NOUT - 1)
            def _():
                pltpu.async_copy(
                    y_hbm.at[src_v.at[c + NBUF]], rows_v.at[b], sems.at[b])

        return carry

    lax.fori_loop(0, NOUT, body, 0)

    plsc.subcore_barrier()
    pltpu.sync_copy(acc.at[pl.ds(sid * ZR, ZR)], stage_v)

    @pl.when(cid == 0)
    def _():
        pltpu.sync_copy(stage_v, p0_hbm.at[pl.ds(sid * ZR, ZR)])

    @pl.when(cid == 1)
    def _():
        pltpu.sync_copy(stage_v, p1_hbm.at[pl.ds(sid * ZR, ZR)])


def kernel(x, edge_index, W1, b1, W2, b2, eps):
    y = pl.pallas_call(
        _project_kernel,
        out_shape=jax.ShapeDtypeStruct((N_PAD, D_HID), jnp.float32),
    )(x, W1)

    e4 = edge_index.reshape(2, NW, NCH, CH)
    p0, p1 = _sc_aggregate(y, e4)

    scale = (1.0 + eps).reshape(1, 1)
    out = pl.pallas_call(
        _mlp_kernel,
        out_shape=jax.ShapeDtypeStruct((N_PAD, D_HID), jnp.float32),
    )(y, p0, p1, W2, b1.reshape(1, D_HID), b2.reshape(1, D_HID), scale)
    return out[:N_NODES]
